# baseline (speedup 1.0000x reference)
import functools

import jax
import jax.numpy as jnp
from jax import lax
from jax.experimental import pallas as pl
from jax.experimental.pallas import tpu as pltpu
from jax.experimental.pallas import tpu_sc as plsc

NC = 2
NS = 16
L = 16
CHUNK = 128


def _make_kernel(B, D):
    NW = NC * NS
    b_per_w = B // NW
    n_chunks = b_per_w // CHUNK

    mesh = plsc.VectorSubcoreMesh(
        core_axis_name="c", subcore_axis_name="s",
        num_cores=NC, num_subcores=NS)

    @functools.partial(
        pl.kernel,
        mesh=mesh,
        out_type=jax.ShapeDtypeStruct((B,), jnp.float32),
        scratch_types=[
            pltpu.VMEM((3 * n_chunks, CHUNK), jnp.int32),
            pltpu.VMEM((b_per_w, D), jnp.float32),
            pltpu.VMEM((b_per_w, D), jnp.float32),
            pltpu.VMEM((b_per_w, D), jnp.float32),
            pltpu.VMEM((b_per_w,), jnp.float32),
        ] + [pltpu.SemaphoreType.DMA] * (b_per_w // CHUNK),
        compiler_params=pltpu.CompilerParams(
            needs_layout_passes=False, use_tc_tiling_on_sc=False),
    )
    def k(idx_hbm, I_hbm, M_hbm, A_hbm, out_hbm,
          idx_v, r0, r1, r2, out_v, *sems):
        wid = lax.axis_index("s") * NC + lax.axis_index("c")
        base = wid * b_per_w

        pltpu.sync_copy(idx_hbm.at[wid], idx_v)

        cps = []
        for j in range(n_chunks):
            dst = pl.ds(j * CHUNK, CHUNK)
            cps.append((
                pltpu.async_copy(I_hbm.at[idx_v.at[j]], r0.at[dst], sems[j]),
                pltpu.async_copy(M_hbm.at[idx_v.at[n_chunks + j]], r1.at[dst], sems[j]),
                pltpu.async_copy(A_hbm.at[idx_v.at[2 * n_chunks + j]], r2.at[dst], sems[j]),
            ))

        lane = lax.iota(jnp.int32, L)

        def block(i, carry):
            ridx = lane + i * L
            acc = jnp.zeros((L,), jnp.float32)
            for d in range(D):
                cd = jnp.full((L,), d, jnp.int32)
                v0 = plsc.load_gather(r0, [ridx, cd])
                v1 = plsc.load_gather(r1, [ridx, cd])
                v2 = plsc.load_gather(r2, [ridx, cd])
                acc = acc + v0 * v1 * v2
            out_v[pl.ds(i * L, L)] = acc
            return carry

        blocks_per_chunk = CHUNK // L
        for j in range(n_chunks):
            for cp in cps[j]:
                cp.wait()
            lax.fori_loop(j * blocks_per_chunk, (j + 1) * blocks_per_chunk,
                          block, 0)

        pltpu.sync_copy(out_v, out_hbm.at[pl.ds(base, b_per_w)])

    return k


def kernel(batch, I, M, A):
    B = batch.shape[0]
    D = I.shape[1]
    NW = NC * NS
    n_chunks = B // NW // CHUNK
    idx = (batch.reshape(NW, n_chunks, CHUNK, 3)
           .transpose(0, 3, 1, 2)
           .reshape(NW, 3 * n_chunks, CHUNK))
    k = _make_kernel(B, D)
    return k(idx, I, M, A)


# speedup vs baseline: 1.4331x; 1.4331x over previous
import functools

import jax
import jax.numpy as jnp
from jax import lax
from jax.experimental import pallas as pl
from jax.experimental.pallas import tpu as pltpu
from jax.experimental.pallas import tpu_sc as plsc

NC = 2
NS = 16
L = 16
CHUNK = 128
IDX_ROWS = 16


def _make_kernel(B, D):
    NW = NC * NS
    b_per_w = B // NW
    n_chunks = b_per_w // CHUNK

    mesh = plsc.VectorSubcoreMesh(
        core_axis_name="c", subcore_axis_name="s",
        num_cores=NC, num_subcores=NS)

    @functools.partial(
        pl.kernel,
        mesh=mesh,
        out_type=jax.ShapeDtypeStruct((B,), jnp.float32),
        scratch_types=[
            pltpu.VMEM((IDX_ROWS, CHUNK), jnp.int32),
            pltpu.VMEM((CHUNK, 16), jnp.float32),
            pltpu.VMEM((CHUNK, 16), jnp.float32),
            pltpu.VMEM((CHUNK, 16), jnp.float32),
            pltpu.VMEM((CHUNK, 16), jnp.float32),
            pltpu.VMEM((CHUNK, 16), jnp.float32),
            pltpu.VMEM((CHUNK, 16), jnp.float32),
            pltpu.VMEM((b_per_w,), jnp.float32),
        ] + [pltpu.SemaphoreType.DMA] * 6,
        compiler_params=pltpu.CompilerParams(needs_layout_passes=False),
    )
    def k(idx_hbm, I_hbm, M_hbm, A_hbm, out_hbm,
          idx_v, bI0, bI1, bM0, bM1, bA0, bA1, out_v, *sems):
        wid = lax.axis_index("s") * NC + lax.axis_index("c")
        base = wid * b_per_w

        pltpu.sync_copy(idx_hbm.at[pl.ds(wid * IDX_ROWS, IDX_ROWS)], idx_v)

        tables = (I_hbm, M_hbm, A_hbm)
        bufs = ((bI0, bI1), (bM0, bM1), (bA0, bA1))

        def fire(c):
            p = c % 2
            for t in range(3):
                def issue(g, carry, t=t, c=c, p=p):
                    vec = idx_v[t * n_chunks + c, pl.ds(g * L, L)]
                    for u in range(L):
                        pltpu.async_copy(
                            tables[t].at[pl.ds(vec[u], 1)],
                            bufs[t][p].at[pl.ds(g * L + u, 1)],
                            sems[t * 2 + p])
                    return carry
                lax.fori_loop(0, CHUNK // L, issue, 0)

        def drain(c):
            p = c % 2
            for t in range(3):
                pltpu.make_async_copy(
                    tables[t].at[pl.ds(0, CHUNK)], bufs[t][p],
                    sems[t * 2 + p]).wait()

        lane = lax.iota(jnp.int32, L)

        fire(0)
        for c in range(n_chunks):
            if c + 1 < n_chunks:
                fire(c + 1)
            drain(c)
            p = c % 2
            b0, b1, b2 = bufs[0][p], bufs[1][p], bufs[2][p]

            def block(i, carry, c=c, b0=b0, b1=b1, b2=b2):
                rsel = lane + i * L
                acc = jnp.zeros((L,), jnp.float32)
                for d in range(D):
                    cd = jnp.full((L,), d, jnp.int32)
                    v0 = plsc.load_gather(b0, [rsel, cd])
                    v1 = plsc.load_gather(b1, [rsel, cd])
                    v2 = plsc.load_gather(b2, [rsel, cd])
                    acc = acc + v0 * v1 * v2
                out_v[pl.ds(c * CHUNK + i * L, L)] = acc
                return carry

            lax.fori_loop(0, CHUNK // L, block, 0)

        pltpu.sync_copy(out_v, out_hbm.at[pl.ds(base, b_per_w)])

    return k


def kernel(batch, I, M, A):
    B = batch.shape[0]
    D = I.shape[1]
    NW = NC * NS
    n_chunks = B // NW // CHUNK
    idx = (batch.reshape(NW, n_chunks, CHUNK, 3)
           .transpose(0, 3, 1, 2)
           .reshape(NW, 3 * n_chunks, CHUNK))
    pad = jnp.zeros((NW, IDX_ROWS - 3 * n_chunks, CHUNK), jnp.int32)
    idx = jnp.concatenate([idx, pad], axis=1).reshape(NW * IDX_ROWS, CHUNK)
    k = _make_kernel(B, D)
    return k(idx, I, M, A)
